# baseline (speedup 1.0000x reference)
import jax
import jax.numpy as jnp
from jax import lax
from jax.experimental import pallas as pl
from jax.experimental.pallas import tpu as pltpu
from jax.experimental.pallas import tpu_sc as plsc

_DIM = 64
_BW = 128
_LB = 8
_NW = 32
_NBLK = 25


def _gather_body(tbl2, idx_hbm, out_hbm,
                 idxv, idx2v, hv64v, gbuf, sbuf, dump,
                 gsem0, gsem1, osem0, osem1):
    gbufs = (gbuf.at[0], gbuf.at[1])
    sbufs = (sbuf.at[0], sbuf.at[1])
    gsems = (gsem0, gsem1)
    osems = (osem0, osem1)

    wid = lax.axis_index("s") * 2 + lax.axis_index("c")
    b0 = wid * _BW

    def load_idx_block(lb):
        pltpu.sync_copy(idx_hbm.at[pl.ds(lb * _LB, _LB), pl.ds(b0, _BW)], idxv)
        def prep(g, carry):
            r = g // 8
            c = (g % 8) * 16
            v = idxv[r, pl.ds(c, 16)]
            idx2v[r, pl.ds(c, 16)] = lax.shift_right_logical(v, 1)
            hv64v[r, pl.ds(c, 16)] = lax.shift_left(
                lax.bitwise_and(v, 1), 6)
            return carry
        lax.fori_loop(0, _LB * 8, prep, 0)

    def fire_gather(r, buf):
        pltpu.async_copy(tbl2.at[idx2v.at[r]], gbufs[buf], gsems[buf])

    def wait_gather(buf):
        pltpu.make_async_copy(tbl2.at[idx2v.at[0]], gbufs[buf],
                              gsems[buf]).wait()

    def fire_out(r, lb, buf):
        pltpu.async_copy(sbufs[buf],
                         out_hbm.at[lb * _LB + r, :, pl.ds(b0, _BW)],
                         osems[buf])

    def wait_out(buf):
        pltpu.make_async_copy(sbufs[buf], dump, osems[buf]).wait()

    def transpose_row(r, buf):
        lane = lax.iota(jnp.int32, 16)
        for g in range(8):
            rows = lane + (16 * g)
            hv = hv64v[r, pl.ds(16 * g, 16)]
            for d in range(_DIM):
                v = plsc.load_gather(gbufs[buf], [rows, hv + d])
                sbufs[buf][d, pl.ds(16 * g, 16)] = v

    pltpu.async_copy(sbufs[0], dump, osems[0])
    pltpu.async_copy(sbufs[1], dump, osems[1])

    def block(lb, carry):
        load_idx_block(lb)
        fire_gather(0, 0)
        fire_gather(1, 1)

        def two_rows(h, carry2):
            for sub in range(2):
                r = 2 * h + sub
                buf = sub
                wait_gather(buf)
                wait_out(buf)
                transpose_row(r, buf)
                fire_out(r, lb, buf)

                @pl.when(r + 2 < _LB)
                def _():
                    fire_gather(r + 2, buf)
            return carry2

        lax.fori_loop(0, _LB // 2, two_rows, 0)
        return carry

    lax.fori_loop(0, _NBLK, block, 0)
    wait_out(0)
    wait_out(1)


def kernel(inputs, table):
    seq, batch = inputs.shape
    vocab = table.shape[0]
    tbl2 = table.reshape(vocab // 2, 2 * _DIM)
    mesh = plsc.VectorSubcoreMesh(core_axis_name="c", subcore_axis_name="s")
    out_t = pl.kernel(
        _gather_body,
        out_type=jax.ShapeDtypeStruct((seq, _DIM, batch), jnp.float32),
        mesh=mesh,
        compiler_params=pltpu.CompilerParams(needs_layout_passes=False),
        scratch_types=[
            pltpu.VMEM((_LB, _BW), jnp.int32),
            pltpu.VMEM((_LB, _BW), jnp.int32),
            pltpu.VMEM((_LB, _BW), jnp.int32),
            pltpu.VMEM((2, _BW, 2 * _DIM), jnp.float32),
            pltpu.VMEM((2, _DIM, _BW), jnp.float32),
            pltpu.HBM((_DIM, _BW), jnp.float32),
            pltpu.SemaphoreType.DMA,
            pltpu.SemaphoreType.DMA,
            pltpu.SemaphoreType.DMA,
            pltpu.SemaphoreType.DMA,
        ],
    )(tbl2, inputs)
    return out_t.transpose(0, 2, 1)


# speedup vs baseline: 1.8229x; 1.8229x over previous
import jax
import jax.numpy as jnp
from jax import lax
from jax.experimental import pallas as pl
from jax.experimental.pallas import tpu as pltpu
from jax.experimental.pallas import tpu_sc as plsc

_DIM = 64
_BW = 128
_LB = 8
_NW = 32
_NBLK = 25


def _gather_body(tbl2, idx_hbm, out_hbm,
                 idxv, idx2v, hv64v, gbuf, sbuf, dump,
                 gsem0, gsem1, osem0, osem1):
    gbufs = (gbuf.at[0], gbuf.at[1])
    sbufs = (sbuf.at[0], sbuf.at[1])
    gsems = (gsem0, gsem1)
    osems = (osem0, osem1)

    wid = lax.axis_index("s") * 2 + lax.axis_index("c")
    b0 = wid * _BW

    def load_idx_block(lb):
        pltpu.sync_copy(idx_hbm.at[pl.ds(lb * _LB, _LB), pl.ds(b0, _BW)], idxv)
        def prep(g, carry):
            r = g // 8
            c = (g % 8) * 16
            v = idxv[r, pl.ds(c, 16)]
            idx2v[r, pl.ds(c, 16)] = lax.shift_right_logical(v, 1)
            hv64v[r, pl.ds(c, 16)] = lax.shift_left(
                lax.bitwise_and(v, 1), 6)
            return carry
        lax.fori_loop(0, _LB * 8, prep, 0)

    def fire_gather(r, buf):
        pltpu.async_copy(tbl2.at[idx2v.at[r]], gbufs[buf], gsems[buf])

    def wait_gather(buf):
        pltpu.make_async_copy(tbl2.at[idx2v.at[0]], gbufs[buf],
                              gsems[buf]).wait()

    def fire_out(r, lb, buf):
        pltpu.async_copy(sbufs[buf],
                         out_hbm.at[lb * _LB + r, :, pl.ds(b0, _BW)],
                         osems[buf])

    def wait_out(buf):
        pltpu.make_async_copy(sbufs[buf], dump, osems[buf]).wait()

    lane = lax.iota(jnp.int32, 16)
    rots = [lax.bitwise_and(lane + t, 15) for t in range(16)]
    rows_g = [lane + 16 * g for g in range(8)]

    def transpose_row(r, buf):
        def gk_step(i, carry):
            g16 = lax.shift_right_logical(i, 2) * 16
            k16 = lax.bitwise_and(i, 3) * 16
            rows = lane + g16
            hv = hv64v[r, pl.ds(g16, 16)]
            colbase = hv + k16
            for t in range(16):
                v = plsc.load_gather(gbufs[buf],
                                     [rows, colbase + rots[t]])
                plsc.store_scatter(sbufs[buf],
                                   [rots[t] + k16, rows], v)
            return carry

        lax.fori_loop(0, 32, gk_step, 0)

    pltpu.async_copy(sbufs[0], dump, osems[0])
    pltpu.async_copy(sbufs[1], dump, osems[1])

    def block(lb, carry):
        load_idx_block(lb)
        fire_gather(0, 0)
        fire_gather(1, 1)

        def two_rows(h, carry2):
            for sub in range(2):
                r = 2 * h + sub
                buf = sub
                wait_gather(buf)
                wait_out(buf)
                transpose_row(r, buf)
                fire_out(r, lb, buf)

                @pl.when(r + 2 < _LB)
                def _():
                    fire_gather(r + 2, buf)
            return carry2

        lax.fori_loop(0, _LB // 2, two_rows, 0)
        return carry

    lax.fori_loop(0, _NBLK, block, 0)
    wait_out(0)
    wait_out(1)


def kernel(inputs, table):
    seq, batch = inputs.shape
    vocab = table.shape[0]
    tbl2 = table.reshape(vocab // 2, 2 * _DIM)
    mesh = plsc.VectorSubcoreMesh(core_axis_name="c", subcore_axis_name="s")
    out_t = pl.kernel(
        _gather_body,
        out_type=jax.ShapeDtypeStruct((seq, _DIM, batch), jnp.float32),
        mesh=mesh,
        compiler_params=pltpu.CompilerParams(needs_layout_passes=False),
        scratch_types=[
            pltpu.VMEM((_LB, _BW), jnp.int32),
            pltpu.VMEM((_LB, _BW), jnp.int32),
            pltpu.VMEM((_LB, _BW), jnp.int32),
            pltpu.VMEM((2, _BW, 2 * _DIM), jnp.float32),
            pltpu.VMEM((2, _DIM, _BW), jnp.float32),
            pltpu.HBM((_DIM, _BW), jnp.float32),
            pltpu.SemaphoreType.DMA,
            pltpu.SemaphoreType.DMA,
            pltpu.SemaphoreType.DMA,
            pltpu.SemaphoreType.DMA,
        ],
    )(tbl2, inputs)
    return out_t.transpose(0, 2, 1)


# speedup vs baseline: 1.9697x; 1.0805x over previous
import jax
import jax.numpy as jnp
from jax import lax
from jax.experimental import pallas as pl
from jax.experimental.pallas import tpu as pltpu
from jax.experimental.pallas import tpu_sc as plsc

_DIM = 64
_BW = 128
_LB = 8
_NW = 32
_NBLK = 25
_NTC = 7812
_HSTEPS = 123


def _repack_body(tt, tail, tbl2, inblk, stg, isem0, isem1, osem0, osem1,
                 dump):
    inblks = (inblk.at[0], inblk.at[1])
    stgs = (stg.at[0], stg.at[1])
    isems = (isem0, isem1)
    osems = (osem0, osem1)

    wid = lax.axis_index("s") * 2 + lax.axis_index("c")

    lane = lax.iota(jnp.int32, 16)
    rots = [lax.bitwise_and(lane + t, 15) for t in range(16)]
    lanehalf = lax.shift_right_logical(lane, 1)
    hvlane = lax.shift_left(lax.bitwise_and(lane, 1), 6)

    def fire_in_full(tc, buf):
        pltpu.async_copy(tt.at[:, pl.ds(tc * 128, 128)], inblks[buf],
                         isems[buf])

    def wait_in_full(buf):
        pltpu.make_async_copy(tt.at[:, pl.ds(0, 128)], inblks[buf],
                              isems[buf]).wait()

    def fire_out(tc, buf):
        pltpu.async_copy(stgs[buf], tbl2.at[pl.ds(tc * 64, 64)], osems[buf])

    def wait_out(buf):
        pltpu.make_async_copy(stgs[buf], dump, osems[buf]).wait()

    def pack(buf, nc0g_pow):
        def combo(i, carry):
            q16 = lax.shift_left(lax.shift_right_logical(i, nc0g_pow), 4)
            c0 = lax.shift_left(lax.bitwise_and(i, (1 << nc0g_pow) - 1), 4)
            cvec = c0 + lane
            rowv = lanehalf + lax.shift_right_logical(c0, 1)
            for t in range(16):
                dvec = q16 + rots[t]
                v = plsc.load_gather(inblks[buf], [dvec, cvec])
                plsc.store_scatter(stgs[buf], [rowv, hvlane + dvec], v)
            return carry
        lax.fori_loop(0, 4 << nc0g_pow, combo, 0)

    pltpu.async_copy(stgs[0], dump, osems[0])
    pltpu.async_copy(stgs[1], dump, osems[1])
    fire_in_full(wid, 0)

    def step(h, carry):
        for sub in range(2):
            j = 2 * h + sub
            buf = sub
            tcn = wid + 32 * (j + 1)
            tc = wid + 32 * j

            @pl.when(tcn < _NTC)
            def _():
                fire_in_full(tcn, 1 - buf)

            @pl.when(tc < _NTC)
            def _():
                wait_in_full(buf)
                wait_out(buf)
                pack(buf, 3)
                fire_out(tc, buf)
        return carry

    lax.fori_loop(0, _HSTEPS, step, 0)

    @pl.when(wid == 31)
    def _():
        pltpu.sync_copy(tail, inblks[0])
        wait_out(0)
        pack(0, 3)
        pltpu.async_copy(stgs[0], tbl2.at[pl.ds(_NTC * 64 - 32, 64)],
                         osems[0])

    wait_out(0)
    wait_out(1)


def _gather_body(tbl2, idx_hbm, out_hbm,
                 idxv, idx2v, hv64v, gbuf, sbuf, dump,
                 gsem0, gsem1, osem0, osem1):
    gbufs = (gbuf.at[0], gbuf.at[1])
    sbufs = (sbuf.at[0], sbuf.at[1])
    gsems = (gsem0, gsem1)
    osems = (osem0, osem1)

    wid = lax.axis_index("s") * 2 + lax.axis_index("c")
    b0 = wid * _BW

    lane = lax.iota(jnp.int32, 16)
    rots = [lax.bitwise_and(lane + t, 15) for t in range(16)]

    def load_idx_block(lb):
        pltpu.sync_copy(idx_hbm.at[pl.ds(lb * _LB, _LB), pl.ds(b0, _BW)], idxv)
        def prep(g, carry):
            r = g // 8
            c = (g % 8) * 16
            v = idxv[r, pl.ds(c, 16)]
            idx2v[r, pl.ds(c, 16)] = lax.shift_right_logical(v, 1)
            hv64v[r, pl.ds(c, 16)] = lax.shift_left(
                lax.bitwise_and(v, 1), 6)
            return carry
        lax.fori_loop(0, _LB * 8, prep, 0)

    def fire_gather(r, buf):
        pltpu.async_copy(tbl2.at[idx2v.at[r]], gbufs[buf], gsems[buf])

    def wait_gather(buf):
        pltpu.make_async_copy(tbl2.at[idx2v.at[0]], gbufs[buf],
                              gsems[buf]).wait()

    def fire_out(r, lb, buf):
        pltpu.async_copy(sbufs[buf],
                         out_hbm.at[lb * _LB + r, :, pl.ds(b0, _BW)],
                         osems[buf])

    def wait_out(buf):
        pltpu.make_async_copy(sbufs[buf], dump, osems[buf]).wait()

    def transpose_row(r, buf):
        def gk_step(i, carry):
            g16 = lax.shift_right_logical(i, 2) * 16
            k16 = lax.bitwise_and(i, 3) * 16
            rows = lane + g16
            hv = hv64v[r, pl.ds(g16, 16)]
            colbase = hv + k16
            for t in range(16):
                v = plsc.load_gather(gbufs[buf],
                                     [rows, colbase + rots[t]])
                plsc.store_scatter(sbufs[buf],
                                   [rots[t] + k16, rows], v)
            return carry

        lax.fori_loop(0, 32, gk_step, 0)

    pltpu.async_copy(sbufs[0], dump, osems[0])
    pltpu.async_copy(sbufs[1], dump, osems[1])

    def block(lb, carry):
        load_idx_block(lb)
        fire_gather(0, 0)
        fire_gather(1, 1)

        def two_rows(h, carry2):
            for sub in range(2):
                r = 2 * h + sub
                buf = sub
                wait_gather(buf)
                wait_out(buf)
                transpose_row(r, buf)
                fire_out(r, lb, buf)

                @pl.when(r + 2 < _LB)
                def _():
                    fire_gather(r + 2, buf)
            return carry2

        lax.fori_loop(0, _LB // 2, two_rows, 0)
        return carry

    lax.fori_loop(0, _NBLK, block, 0)
    wait_out(0)
    wait_out(1)


def kernel(inputs, table):
    seq, batch = inputs.shape
    vocab = table.shape[0]
    mesh = plsc.VectorSubcoreMesh(core_axis_name="c", subcore_axis_name="s")

    tt = jnp.transpose(table)
    tail = jnp.transpose(lax.slice(table, (vocab - 128, 0), (vocab, _DIM)))
    tbl2 = pl.kernel(
        _repack_body,
        out_type=jax.ShapeDtypeStruct((vocab // 2, 2 * _DIM), jnp.float32),
        mesh=mesh,
        compiler_params=pltpu.CompilerParams(needs_layout_passes=False),
        scratch_types=[
            pltpu.VMEM((2, _DIM, 128), jnp.float32),
            pltpu.VMEM((2, _DIM, 128), jnp.float32),
            pltpu.SemaphoreType.DMA,
            pltpu.SemaphoreType.DMA,
            pltpu.SemaphoreType.DMA,
            pltpu.SemaphoreType.DMA,
            pltpu.HBM((_DIM, 128), jnp.float32),
        ],
    )(tt, tail)

    out_t = pl.kernel(
        _gather_body,
        out_type=jax.ShapeDtypeStruct((seq, _DIM, batch), jnp.float32),
        mesh=mesh,
        compiler_params=pltpu.CompilerParams(needs_layout_passes=False),
        scratch_types=[
            pltpu.VMEM((_LB, _BW), jnp.int32),
            pltpu.VMEM((_LB, _BW), jnp.int32),
            pltpu.VMEM((_LB, _BW), jnp.int32),
            pltpu.VMEM((2, _BW, 2 * _DIM), jnp.float32),
            pltpu.VMEM((2, _DIM, _BW), jnp.float32),
            pltpu.HBM((_DIM, _BW), jnp.float32),
            pltpu.SemaphoreType.DMA,
            pltpu.SemaphoreType.DMA,
            pltpu.SemaphoreType.DMA,
            pltpu.SemaphoreType.DMA,
        ],
    )(tbl2, inputs)
    return out_t.transpose(0, 2, 1)
